# baseline (speedup 1.0000x reference)
import jax
import jax.numpy as jnp
from jax.experimental import pallas as pl
from jax.experimental.pallas import tpu as pltpu

CODEBOOK_SIZE = 128
EMBEDDING_DIM = 32
BATCH = 256
SEQ = 576
PACK = 128 // EMBEDDING_DIM
SEQP = SEQ // PACK

BATCH_PER_STEP = 8
ROWS = BATCH_PER_STEP * SEQP
GRID = BATCH // BATCH_PER_STEP


def _vq_body(lat_ref, cb_ref, bstack_ref, wstack_ref, upper_ref, cbflat_ref,
             policy_ref, quant_ref, cbset_ref):
    lat = lat_ref[...].reshape(ROWS, 128)
    cb = cb_ref[...]
    cb_norm = jnp.sum(cb * cb, axis=1)[None, :]
    upper = upper_ref[...]
    latsq = lat * lat
    q = jnp.zeros((ROWS, 128), jnp.float32)
    for j in range(PACK):
        mm_j = jnp.dot(lat, bstack_ref[j], preferred_element_type=jnp.float32)
        ln_j = jnp.sum(
            latsq[:, j * EMBEDDING_DIM:(j + 1) * EMBEDDING_DIM],
            axis=1, keepdims=True,
        )
        scores_j = (ln_j + cb_norm) - 2.0 * mm_j
        smin_j = jnp.min(scores_j, axis=1, keepdims=True)
        eq_j = (scores_j == smin_j).astype(jnp.float32)
        prefix_j = jnp.dot(eq_j, upper, preferred_element_type=jnp.float32)
        onehot_j = eq_j * jnp.maximum(1.0 - prefix_j, 0.0)
        q = q + jnp.dot(onehot_j, wstack_ref[j],
                        preferred_element_type=jnp.float32)
    shape3 = (BATCH_PER_STEP, SEQP, 128)
    quant_ref[...] = q.reshape(shape3)
    policy_ref[...] = (lat + (q - lat)).reshape(shape3)
    cbset_ref[...] = jnp.broadcast_to(
        cbflat_ref[...][None], (BATCH_PER_STEP, EMBEDDING_DIM, 128)
    )


def kernel(latent, codebook):
    latp = latent.reshape(BATCH, SEQP, 128)
    cbt = jnp.swapaxes(codebook, 0, 1)
    cbflat = codebook.reshape(EMBEDDING_DIM, 128)
    sub = jnp.arange(128, dtype=jnp.int32) // EMBEDDING_DIM
    jidx = jnp.arange(PACK, dtype=jnp.int32)
    bstack = jnp.where(
        sub[None, :, None] == jidx[:, None, None],
        jnp.tile(cbt, (PACK, 1))[None], 0.0,
    )
    wstack = jnp.where(
        sub[None, None, :] == jidx[:, None, None],
        jnp.tile(codebook, (1, PACK))[None], 0.0,
    )
    k = jnp.arange(128, dtype=jnp.int32)
    upper = (k[:, None] < k[None, :]).astype(jnp.float32)
    policy, quant, cbset = pl.pallas_call(
        _vq_body,
        grid=(GRID,),
        in_specs=[
            pl.BlockSpec((BATCH_PER_STEP, SEQP, 128), lambda i: (i, 0, 0)),
            pl.BlockSpec((CODEBOOK_SIZE, EMBEDDING_DIM), lambda i: (0, 0)),
            pl.BlockSpec((PACK, 128, 128), lambda i: (0, 0, 0)),
            pl.BlockSpec((PACK, 128, 128), lambda i: (0, 0, 0)),
            pl.BlockSpec((128, 128), lambda i: (0, 0)),
            pl.BlockSpec((EMBEDDING_DIM, 128), lambda i: (0, 0)),
        ],
        out_specs=[
            pl.BlockSpec((BATCH_PER_STEP, SEQP, 128), lambda i: (i, 0, 0)),
            pl.BlockSpec((BATCH_PER_STEP, SEQP, 128), lambda i: (i, 0, 0)),
            pl.BlockSpec(
                (BATCH_PER_STEP, EMBEDDING_DIM, 128), lambda i: (i, 0, 0)
            ),
        ],
        out_shape=[
            jax.ShapeDtypeStruct((BATCH, SEQP, 128), jnp.float32),
            jax.ShapeDtypeStruct((BATCH, SEQP, 128), jnp.float32),
            jax.ShapeDtypeStruct((BATCH, EMBEDDING_DIM, 128), jnp.float32),
        ],
        compiler_params=pltpu.CompilerParams(
            dimension_semantics=("parallel",),
        ),
    )(latp, codebook, bstack, wstack, upper, cbflat)
    shape3 = (BATCH, SEQ, EMBEDDING_DIM)
    return (
        policy.reshape(shape3),
        quant.reshape(shape3),
        cbset.reshape(BATCH, CODEBOOK_SIZE, EMBEDDING_DIM),
    )


# speedup vs baseline: 1.2921x; 1.2921x over previous
import jax
import jax.numpy as jnp
from jax.experimental import pallas as pl
from jax.experimental.pallas import tpu as pltpu

CODEBOOK_SIZE = 128
EMBEDDING_DIM = 32
BATCH = 256
SEQ = 576
PACK = 128 // EMBEDDING_DIM
SEQP = SEQ // PACK

BATCH_PER_STEP = 16
ROWS = BATCH_PER_STEP * SEQP
GRID = BATCH // BATCH_PER_STEP


def _vq_body(lat_ref, cb_ref, bstack_ref, wstack_ref, iota_ref, cbflat_ref,
             policy_ref, quant_ref, cbset_ref):
    lat = lat_ref[...].reshape(ROWS, 128)
    cb = cb_ref[...]
    cb_norm = jnp.sum(cb * cb, axis=1)[None, :]
    iota = jnp.broadcast_to(iota_ref[0:1, :], (ROWS, 128))
    latsq = lat * lat
    q = jnp.zeros((ROWS, 128), jnp.float32)
    for j in range(PACK):
        mm_j = jnp.dot(lat, bstack_ref[j], preferred_element_type=jnp.float32)
        ln_j = jnp.sum(
            latsq[:, j * EMBEDDING_DIM:(j + 1) * EMBEDDING_DIM],
            axis=1, keepdims=True,
        )
        scores_j = (ln_j + cb_norm) - 2.0 * mm_j
        smin_j = jnp.min(scores_j, axis=1, keepdims=True)
        idx_j = jnp.min(
            jnp.where(scores_j == smin_j, iota, float(CODEBOOK_SIZE)),
            axis=1, keepdims=True,
        )
        onehot_j = (iota == idx_j).astype(jnp.float32)
        q = q + jnp.dot(onehot_j, wstack_ref[j],
                        preferred_element_type=jnp.float32)
    shape3 = (BATCH_PER_STEP, SEQP, 128)
    quant_ref[...] = q.reshape(shape3)
    policy_ref[...] = (lat + (q - lat)).reshape(shape3)
    cbset_ref[...] = jnp.broadcast_to(
        cbflat_ref[...][None], (BATCH_PER_STEP, EMBEDDING_DIM, 128)
    )


def kernel(latent, codebook):
    latp = latent.reshape(BATCH, SEQP, 128)
    cbt = jnp.swapaxes(codebook, 0, 1)
    cbflat = codebook.reshape(EMBEDDING_DIM, 128)
    sub = jnp.arange(128, dtype=jnp.int32) // EMBEDDING_DIM
    jidx = jnp.arange(PACK, dtype=jnp.int32)
    bstack = jnp.where(
        sub[None, :, None] == jidx[:, None, None],
        jnp.tile(cbt, (PACK, 1))[None], 0.0,
    )
    wstack = jnp.where(
        sub[None, None, :] == jidx[:, None, None],
        jnp.tile(codebook, (1, PACK))[None], 0.0,
    )
    iota8 = jnp.broadcast_to(
        jnp.arange(128, dtype=jnp.float32)[None, :], (8, 128)
    )
    policy, quant, cbset = pl.pallas_call(
        _vq_body,
        grid=(GRID,),
        in_specs=[
            pl.BlockSpec((BATCH_PER_STEP, SEQP, 128), lambda i: (i, 0, 0)),
            pl.BlockSpec((CODEBOOK_SIZE, EMBEDDING_DIM), lambda i: (0, 0)),
            pl.BlockSpec((PACK, 128, 128), lambda i: (0, 0, 0)),
            pl.BlockSpec((PACK, 128, 128), lambda i: (0, 0, 0)),
            pl.BlockSpec((8, 128), lambda i: (0, 0)),
            pl.BlockSpec((EMBEDDING_DIM, 128), lambda i: (0, 0)),
        ],
        out_specs=[
            pl.BlockSpec((BATCH_PER_STEP, SEQP, 128), lambda i: (i, 0, 0)),
            pl.BlockSpec((BATCH_PER_STEP, SEQP, 128), lambda i: (i, 0, 0)),
            pl.BlockSpec(
                (BATCH_PER_STEP, EMBEDDING_DIM, 128), lambda i: (i, 0, 0)
            ),
        ],
        out_shape=[
            jax.ShapeDtypeStruct((BATCH, SEQP, 128), jnp.float32),
            jax.ShapeDtypeStruct((BATCH, SEQP, 128), jnp.float32),
            jax.ShapeDtypeStruct((BATCH, EMBEDDING_DIM, 128), jnp.float32),
        ],
        compiler_params=pltpu.CompilerParams(
            dimension_semantics=("parallel",),
        ),
    )(latp, codebook, bstack, wstack, iota8, cbflat)
    shape3 = (BATCH, SEQ, EMBEDDING_DIM)
    return (
        policy.reshape(shape3),
        quant.reshape(shape3),
        cbset.reshape(BATCH, CODEBOOK_SIZE, EMBEDDING_DIM),
    )


# speedup vs baseline: 1.3200x; 1.0215x over previous
import jax
import jax.numpy as jnp
from jax.experimental import pallas as pl
from jax.experimental.pallas import tpu as pltpu

CODEBOOK_SIZE = 128
EMBEDDING_DIM = 32
BATCH = 256
SEQ = 576
PACK = 128 // EMBEDDING_DIM
SEQP = SEQ // PACK

BATCH_PER_STEP = 32
ROWS = BATCH_PER_STEP * SEQP
GRID = BATCH // BATCH_PER_STEP


def _vq_body(lat_ref, cb_ref, bstack_ref, wstack_ref, iota_ref, cbflat_ref,
             policy_ref, quant_ref, cbset_ref):
    lat = lat_ref[...].reshape(ROWS, 128)
    cb = cb_ref[...]
    cb_norm = jnp.sum(cb * cb, axis=1)[None, :]
    iota = jnp.broadcast_to(iota_ref[0:1, :], (ROWS, 128))
    latsq = lat * lat
    q = jnp.zeros((ROWS, 128), jnp.float32)
    for j in range(PACK):
        mm_j = jnp.dot(lat, bstack_ref[j], preferred_element_type=jnp.float32)
        ln_j = jnp.sum(
            latsq[:, j * EMBEDDING_DIM:(j + 1) * EMBEDDING_DIM],
            axis=1, keepdims=True,
        )
        scores_j = (ln_j + cb_norm) - 2.0 * mm_j
        smin_j = jnp.min(scores_j, axis=1, keepdims=True)
        idx_j = jnp.min(
            jnp.where(scores_j == smin_j, iota, float(CODEBOOK_SIZE)),
            axis=1, keepdims=True,
        )
        onehot_j = (iota == idx_j).astype(jnp.float32)
        q = q + jnp.dot(onehot_j, wstack_ref[j],
                        preferred_element_type=jnp.float32)
    shape3 = (BATCH_PER_STEP, SEQP, 128)
    quant_ref[...] = q.reshape(shape3)
    policy_ref[...] = (lat + (q - lat)).reshape(shape3)
    cbset_ref[...] = jnp.broadcast_to(
        cbflat_ref[...][None], (BATCH_PER_STEP, EMBEDDING_DIM, 128)
    )


def kernel(latent, codebook):
    latp = latent.reshape(BATCH, SEQP, 128)
    cbt = jnp.swapaxes(codebook, 0, 1)
    cbflat = codebook.reshape(EMBEDDING_DIM, 128)
    sub = jnp.arange(128, dtype=jnp.int32) // EMBEDDING_DIM
    jidx = jnp.arange(PACK, dtype=jnp.int32)
    bstack = jnp.where(
        sub[None, :, None] == jidx[:, None, None],
        jnp.tile(cbt, (PACK, 1))[None], 0.0,
    )
    wstack = jnp.where(
        sub[None, None, :] == jidx[:, None, None],
        jnp.tile(codebook, (1, PACK))[None], 0.0,
    )
    iota8 = jnp.broadcast_to(
        jnp.arange(128, dtype=jnp.float32)[None, :], (8, 128)
    )
    policy, quant, cbset = pl.pallas_call(
        _vq_body,
        grid=(GRID,),
        in_specs=[
            pl.BlockSpec((BATCH_PER_STEP, SEQP, 128), lambda i: (i, 0, 0)),
            pl.BlockSpec((CODEBOOK_SIZE, EMBEDDING_DIM), lambda i: (0, 0)),
            pl.BlockSpec((PACK, 128, 128), lambda i: (0, 0, 0)),
            pl.BlockSpec((PACK, 128, 128), lambda i: (0, 0, 0)),
            pl.BlockSpec((8, 128), lambda i: (0, 0)),
            pl.BlockSpec((EMBEDDING_DIM, 128), lambda i: (0, 0)),
        ],
        out_specs=[
            pl.BlockSpec((BATCH_PER_STEP, SEQP, 128), lambda i: (i, 0, 0)),
            pl.BlockSpec((BATCH_PER_STEP, SEQP, 128), lambda i: (i, 0, 0)),
            pl.BlockSpec(
                (BATCH_PER_STEP, EMBEDDING_DIM, 128), lambda i: (i, 0, 0)
            ),
        ],
        out_shape=[
            jax.ShapeDtypeStruct((BATCH, SEQP, 128), jnp.float32),
            jax.ShapeDtypeStruct((BATCH, SEQP, 128), jnp.float32),
            jax.ShapeDtypeStruct((BATCH, EMBEDDING_DIM, 128), jnp.float32),
        ],
        compiler_params=pltpu.CompilerParams(
            dimension_semantics=("parallel",),
        ),
    )(latp, codebook, bstack, wstack, iota8, cbflat)
    shape3 = (BATCH, SEQ, EMBEDDING_DIM)
    return (
        policy.reshape(shape3),
        quant.reshape(shape3),
        cbset.reshape(BATCH, CODEBOOK_SIZE, EMBEDDING_DIM),
    )
